# baseline (speedup 1.0000x reference)
import functools

import jax
import jax.numpy as jnp
from jax import lax
from jax.experimental import pallas as pl
from jax.experimental.pallas import tpu as pltpu
from jax.experimental.pallas import tpu_sc as plsc

VOCAB = 8192
D = 8192
D2 = D // 2
N = 8192
NC, NS = 2, 16
NW = NC * NS
VPW = VOCAB // NW
R = 4
MAXCH = VPW // R
NBUF = 2
TOKCAP = N + 16
SLOTCAP = VPW + 16
L16 = 16


def _body(table_hbm, idx_hbm, out_hbm, idx_all, tok, stok,
          cnt, gs, scur, uval, duval, buf0, buf1, gsem0, gsem1, ssem):
    wid = lax.axis_index("s") * NC + lax.axis_index("c")
    lo = wid * VPW

    bufs = (buf0, buf1)
    gsems = (gsem0, gsem1)

    iota = lax.iota(jnp.int32, L16)
    zeros = jnp.zeros((L16,), jnp.int32)
    ones = jnp.ones((L16,), jnp.int32)
    lo16 = jnp.full((L16,), lo, jnp.int32)
    hi16 = jnp.full((L16,), lo + VPW, jnp.int32)

    def excl_positions(msk):
        mi = jnp.where(msk, ones, zeros)
        cum = plsc.cumsum(mi)
        return cum - mi, jnp.sum(mi)

    def sload(ref, i):
        v = plsc.load_gather(ref, [jnp.full((L16,), i, jnp.int32)])
        return jnp.sum(jnp.where(iota == 0, v, zeros))

    pltpu.sync_copy(idx_hbm, idx_all)
    for k in range(VPW // L16):
        cnt[pl.ds(k * L16, L16)] = zeros
    for k in range(SLOTCAP // L16):
        uval[pl.ds(k * L16, L16)] = zeros

    def scan(j, m_count):
        v = idx_all[pl.ds(j * L16, L16)]
        msk = jnp.logical_and(v >= lo16, v < hi16)
        pos = j * L16 + iota
        rank, nhit = excl_positions(msk)
        plsc.store_scatter(tok, [m_count + rank], pos, mask=msk)
        return m_count + nhit

    M = lax.fori_loop(0, N // L16, scan, 0)
    m16 = jnp.full((L16,), M, jnp.int32)

    for k in range(SLOTCAP // L16):
        gs[pl.ds(k * L16, L16)] = m16

    ntv = lax.div(M + (L16 - 1), L16)

    def owned(g):
        active = (g * L16 + iota) < m16
        t16 = jnp.where(active, tok[pl.ds(g * L16, L16)], zeros)
        v16 = plsc.load_gather(idx_all, [t16]) - lo16
        v16 = jnp.where(active, v16, zeros)
        return t16, v16, active

    def hist(g, carry):
        _, v16, active = owned(g)
        for l in range(L16):
            ml = jnp.logical_and(active, iota == l)
            plsc.addupdate_scatter(cnt, [v16], ones, mask=ml)
        return carry

    lax.fori_loop(0, ntv, hist, 0)

    def prefix(k, carry):
        running, slotc = carry
        c16 = cnt[pl.ds(k * L16, L16)]
        excl = plsc.cumsum(c16) - c16 + jnp.full((L16,), running, jnp.int32)
        scur[pl.ds(k * L16, L16)] = excl
        present = c16 > 0
        rank, npres = excl_positions(present)
        slotpos = slotc + rank
        plsc.store_scatter(gs, [slotpos], excl, mask=present)
        plsc.store_scatter(uval, [slotpos], lo16 + k * L16 + iota, mask=present)
        return running + jnp.sum(c16), slotc + npres

    _, K = lax.fori_loop(0, VPW // L16, prefix, (0, 0))

    def place(g, carry):
        t16, v16, active = owned(g)
        for l in range(L16):
            ml = jnp.logical_and(active, iota == l)
            p16 = plsc.load_gather(scur, [v16])
            plsc.store_scatter(stok, [p16], t16, mask=ml)
            plsc.addupdate_scatter(scur, [v16], ones, mask=ml)
        return carry

    lax.fori_loop(0, ntv, place, 0)

    for k in range(SLOTCAP // L16):
        u16 = uval[pl.ds(k * L16, L16)]
        base = 2 * (k * L16 + iota)
        plsc.store_scatter(duval, [base], 2 * u16)
        plsc.store_scatter(duval, [base + ones], 2 * u16 + ones)

    nchunk = lax.div(K + (R - 1), R)

    def start_gather(c, b):
        pltpu.make_async_copy(
            table_hbm.at[duval.at[pl.ds(c * 2 * R, 2 * R)]], bufs[b], gsems[b]
        ).start()

    def wait_gather(b):
        pltpu.make_async_copy(
            table_hbm.at[duval.at[pl.ds(0, 2 * R)]], bufs[b], gsems[b]
        ).wait()

    def wait_one_row(q, carry):
        pltpu.make_async_copy(
            buf0.at[pl.ds(0, 2)], out_hbm.at[pl.ds(0, 2)], ssem
        ).wait()
        return carry

    for b in range(NBUF):
        pl.when(b < nchunk)(lambda b=b: start_gather(b, b))

    def chunk(c, b):
        def run():
            wait_gather(b)
            for r in range(R):
                j = c * R + r
                s0 = sload(gs, j)
                s1 = sload(gs, j + 1)

                def put(p, carry):
                    t = sload(stok, p)
                    pltpu.make_async_copy(
                        bufs[b].at[pl.ds(2 * r, 2)],
                        out_hbm.at[pl.ds(2 * t, 2)],
                        ssem,
                    ).start()
                    return carry

                lax.fori_loop(s0, s1, put, 0)
            total = sload(gs, c * R + R) - sload(gs, c * R)
            lax.fori_loop(0, total, wait_one_row, 0)
            pl.when(c + NBUF < nchunk)(lambda: start_gather(c + NBUF, b))

        pl.when(c < nchunk)(run)

    def outer(g, carry):
        for b in range(NBUF):
            chunk(g * NBUF + b, b)
        return carry

    lax.fori_loop(0, MAXCH // NBUF, outer, 0)


@functools.partial(jax.jit, static_argnames=())
def kernel(idx, embedding):
    B, L = idx.shape
    idx_flat = idx.reshape(N).astype(jnp.int32)
    table2 = embedding.reshape(2 * VOCAB, D2)

    mesh = plsc.VectorSubcoreMesh(
        core_axis_name="c", subcore_axis_name="s", num_cores=NC, num_subcores=NS
    )
    out = pl.kernel(
        _body,
        out_type=jax.ShapeDtypeStruct((2 * N, D2), jnp.float32),
        mesh=mesh,
        compiler_params=pltpu.CompilerParams(needs_layout_passes=False),
        scratch_types=[
            pltpu.VMEM((N,), jnp.int32),
            pltpu.VMEM((TOKCAP,), jnp.int32),
            pltpu.VMEM((TOKCAP,), jnp.int32),
            pltpu.VMEM((VPW,), jnp.int32),
            pltpu.VMEM((SLOTCAP,), jnp.int32),
            pltpu.VMEM((VPW,), jnp.int32),
            pltpu.VMEM((SLOTCAP,), jnp.int32),
            pltpu.VMEM((2 * SLOTCAP,), jnp.int32),
            pltpu.VMEM((2 * R, D2), jnp.float32),
            pltpu.VMEM((2 * R, D2), jnp.float32),
            pltpu.SemaphoreType.DMA,
            pltpu.SemaphoreType.DMA,
            pltpu.SemaphoreType.DMA,
        ],
    )(table2, idx_flat)
    return out.reshape(B, L, D)


# speedup vs baseline: 3.6199x; 3.6199x over previous
import functools

import jax
import jax.numpy as jnp
from jax import lax
from jax.experimental import pallas as pl
from jax.experimental.pallas import tpu as pltpu
from jax.experimental.pallas import tpu_sc as plsc

VOCAB = 8192
D = 8192
N = 8192
NC, NS = 2, 16
NW = NC * NS
TPW = N // NW
R = 4
STEPS = TPW // R
NBUF = 2


def _body(table_hbm, idx_hbm, out_hbm, idx_v, buf0, buf1, sem0, sem1):
    wid = lax.axis_index("s") * NC + lax.axis_index("c")
    base = wid * TPW

    pltpu.sync_copy(idx_hbm.at[wid], idx_v)

    bufs = (buf0, buf1)
    sems = (sem0, sem1)

    def start_gather(s, b):
        pltpu.make_async_copy(table_hbm.at[idx_v.at[s]], bufs[b], sems[b]).start()

    def wait_gather(b):
        pltpu.make_async_copy(table_hbm.at[idx_v.at[0]], bufs[b], sems[b]).wait()

    def put(s, b):
        pltpu.sync_copy(bufs[b], out_hbm.at[pl.ds(base + s * R, R)])

    for b in range(NBUF):
        start_gather(b, b)

    def outer(g, carry):
        for b in range(NBUF):
            s = g * NBUF + b
            wait_gather(b)
            put(s, b)
            start_gather(s + NBUF, b)
        return carry

    lax.fori_loop(0, STEPS // NBUF - 1, outer, 0)

    for b in range(NBUF):
        s = STEPS - NBUF + b
        wait_gather(b)
        put(s, b)


@functools.partial(jax.jit, static_argnames=())
def kernel(idx, embedding):
    B, L = idx.shape
    idx3 = idx.reshape(NW, STEPS, R).astype(jnp.int32)

    mesh = plsc.VectorSubcoreMesh(
        core_axis_name="c", subcore_axis_name="s", num_cores=NC, num_subcores=NS
    )
    out = pl.kernel(
        _body,
        out_type=jax.ShapeDtypeStruct((N, D), jnp.float32),
        mesh=mesh,
        scratch_types=[
            pltpu.VMEM((STEPS, R), jnp.int32),
            pltpu.VMEM((R, D), jnp.float32),
            pltpu.VMEM((R, D), jnp.float32),
            pltpu.SemaphoreType.DMA,
            pltpu.SemaphoreType.DMA,
        ],
    )(embedding, idx3)
    return out.reshape(B, L, D)


# speedup vs baseline: 3.6286x; 1.0024x over previous
import functools

import jax
import jax.numpy as jnp
from jax import lax
from jax.experimental import pallas as pl
from jax.experimental.pallas import tpu as pltpu
from jax.experimental.pallas import tpu_sc as plsc

VOCAB = 8192
D = 8192
N = 8192
NC, NS = 2, 16
NW = NC * NS
TPW = N // NW
R = 4
STEPS = TPW // R
NBUF = 2


def _body(table_hbm, idx_hbm, out_hbm, idx_v, buf0, buf1, sem0, sem1):
    wid = lax.axis_index("s") * NC + lax.axis_index("c")
    base = wid * TPW

    pltpu.sync_copy(idx_hbm.at[wid], idx_v)

    bufs = (buf0, buf1)
    sems = (sem0, sem1)

    def start_gather(s, b):
        pltpu.make_async_copy(table_hbm.at[idx_v.at[s]], bufs[b], sems[b]).start()

    def wait_gather(b):
        pltpu.make_async_copy(table_hbm.at[idx_v.at[0]], bufs[b], sems[b]).wait()

    def put(s, b):
        pltpu.sync_copy(bufs[b], out_hbm.at[pl.ds(base + s * R, R)])

    for b in range(NBUF):
        start_gather(b, b)

    def outer(g, carry):
        for b in range(NBUF):
            s = g * NBUF + b
            wait_gather(b)
            put(s, b)
            start_gather(s + NBUF, b)
        return carry

    lax.fori_loop(0, STEPS // NBUF - 1, outer, 0)

    for b in range(NBUF):
        s = STEPS - NBUF + b
        wait_gather(b)
        put(s, b)


@functools.partial(jax.jit, static_argnames=())
def kernel(idx, embedding):
    B, L = idx.shape
    idx3 = idx.reshape(NW, STEPS, R).astype(jnp.int32)

    mesh = plsc.VectorSubcoreMesh(
        core_axis_name="c", subcore_axis_name="s", num_cores=NC, num_subcores=NS
    )
    out = pl.kernel(
        _body,
        out_type=jax.ShapeDtypeStruct((N, D), jnp.float32),
        mesh=mesh,
        compiler_params=pltpu.CompilerParams(
            disable_bounds_checks=True, disable_semaphore_checks=True
        ),
        scratch_types=[
            pltpu.VMEM((STEPS, R), jnp.int32),
            pltpu.VMEM((R, D), jnp.float32),
            pltpu.VMEM((R, D), jnp.float32),
            pltpu.SemaphoreType.DMA,
            pltpu.SemaphoreType.DMA,
        ],
    )(embedding, idx3)
    return out.reshape(B, L, D)
